# baseline (device time: 333964 ns/iter reference)
import jax
import jax.numpy as jnp
from jax import lax
from jax.experimental import pallas as pl
from jax.experimental.pallas import tpu as pltpu

N_DEV = 16
M = 2048
N = 2048
CH = M // N_DEV


def kernel(A, B):
    A = A.astype(jnp.bfloat16)
    B = B.astype(jnp.bfloat16)

    def body(a_ref, b_ref, out_ref, acc_ref, rbuf_ref, send_sems, recv_sems,
             credit_sem):
        my = lax.axis_index("i")
        left = lax.rem(my - 1 + N_DEV, N_DEV)
        right = lax.rem(my + 1, N_DEV)

        barrier = pltpu.get_barrier_semaphore()
        for nbr in (left, right):
            pl.semaphore_signal(
                barrier, inc=1, device_id=(nbr,),
                device_id_type=pl.DeviceIdType.MESH,
            )
        pl.semaphore_wait(barrier, 2)

        acc_ref[...] = jnp.dot(
            a_ref[...], b_ref[...], preferred_element_type=jnp.float32
        )

        for h in range(N_DEV - 1):
            slot = h % 2
            send_idx = lax.rem(my - h + 2 * N_DEV, N_DEV)
            recv_idx = lax.rem(my - h - 1 + 2 * N_DEV, N_DEV)
            if h >= 2:
                pl.semaphore_wait(credit_sem, 1)
            rdma = pltpu.make_async_remote_copy(
                src_ref=acc_ref.at[pl.ds(send_idx * CH, CH), :],
                dst_ref=rbuf_ref.at[slot],
                send_sem=send_sems.at[slot],
                recv_sem=recv_sems.at[slot],
                device_id=(right,),
                device_id_type=pl.DeviceIdType.MESH,
            )
            rdma.start()
            rdma.wait()
            pl.semaphore_signal(
                credit_sem, inc=1, device_id=(left,),
                device_id_type=pl.DeviceIdType.MESH,
            )
            acc_ref[pl.ds(recv_idx * CH, CH), :] = (
                acc_ref[pl.ds(recv_idx * CH, CH), :] + rbuf_ref[slot]
            )
        pl.semaphore_wait(credit_sem, 2)

        o = lax.rem(my + 1, N_DEV)
        z = acc_ref[pl.ds(o * CH, CH), :]
        out_ref[pl.ds(o * CH, CH), :] = (
            z * (1.0 / (1.0 + jnp.exp(-z)))
        ).astype(out_ref.dtype)

        for h in range(N_DEV - 1):
            slot = h % 2
            send_idx = lax.rem(my + 1 - h + 2 * N_DEV, N_DEV)
            if h >= 2:
                pl.semaphore_wait(credit_sem, 1)
            rdma = pltpu.make_async_remote_copy(
                src_ref=out_ref.at[pl.ds(send_idx * CH, CH), :],
                dst_ref=out_ref.at[pl.ds(send_idx * CH, CH), :],
                send_sem=send_sems.at[slot],
                recv_sem=recv_sems.at[slot],
                device_id=(right,),
                device_id_type=pl.DeviceIdType.MESH,
            )
            rdma.start()
            rdma.wait()
            pl.semaphore_signal(
                credit_sem, inc=1, device_id=(left,),
                device_id_type=pl.DeviceIdType.MESH,
            )
        pl.semaphore_wait(credit_sem, 2)

    return pl.pallas_call(
        body,
        out_shape=jax.ShapeDtypeStruct((M, N), jnp.bfloat16),
        in_specs=[
            pl.BlockSpec(memory_space=pltpu.VMEM),
            pl.BlockSpec(memory_space=pltpu.VMEM),
        ],
        out_specs=pl.BlockSpec(memory_space=pltpu.VMEM),
        scratch_shapes=[
            pltpu.VMEM((M, N), jnp.float32),
            pltpu.VMEM((2, CH, N), jnp.float32),
            pltpu.SemaphoreType.DMA((2,)),
            pltpu.SemaphoreType.DMA((2,)),
            pltpu.SemaphoreType.REGULAR,
        ],
        compiler_params=pltpu.CompilerParams(collective_id=0),
    )(A, B)


# device time: 208527 ns/iter; 1.6015x vs baseline; 1.6015x over previous
import jax
import jax.numpy as jnp
from jax import lax
from jax.experimental import pallas as pl
from jax.experimental.pallas import tpu as pltpu

N_DEV = 16
M = 2048
N = 2048
CH = M // N_DEV
HN = N // 2

MESH = pl.DeviceIdType.MESH


def kernel(A, B):
    A = A.astype(jnp.bfloat16)
    B = B.astype(jnp.bfloat16)

    def body(a_ref, b_ref, out_ref, acc_ref,
             sbuf_r, rbuf_r, sbuf_l, rbuf_l,
             send_sems_r, recv_sems_r, send_sems_l, recv_sems_l,
             credit_r, credit_l):
        my = lax.axis_index("i")
        left = lax.rem(my - 1 + N_DEV, N_DEV)
        right = lax.rem(my + 1, N_DEV)

        barrier = pltpu.get_barrier_semaphore()
        for nbr in (left, right):
            pl.semaphore_signal(barrier, inc=1, device_id=(nbr,),
                                device_id_type=MESH)
        pl.semaphore_wait(barrier, 2)

        acc_ref[...] = jnp.dot(
            a_ref[...], b_ref[...], preferred_element_type=jnp.float32
        )

        sbuf_r[0] = acc_ref[pl.ds(my * CH, CH), pl.ds(0, HN)].astype(jnp.bfloat16)
        sbuf_l[0] = acc_ref[pl.ds(my * CH, CH), pl.ds(HN, HN)].astype(jnp.bfloat16)
        for h in range(N_DEV - 1):
            slot = h % 2
            nslot = (h + 1) % 2
            recv_r = lax.rem(my - h - 1 + 2 * N_DEV, N_DEV)
            recv_l = lax.rem(my + h + 1, N_DEV)
            if h >= 2:
                pl.semaphore_wait(credit_r, 1)
                pl.semaphore_wait(credit_l, 1)
            rdma_r = pltpu.make_async_remote_copy(
                src_ref=sbuf_r.at[slot], dst_ref=rbuf_r.at[slot],
                send_sem=send_sems_r.at[slot], recv_sem=recv_sems_r.at[slot],
                device_id=(right,), device_id_type=MESH,
            )
            rdma_l = pltpu.make_async_remote_copy(
                src_ref=sbuf_l.at[slot], dst_ref=rbuf_l.at[slot],
                send_sem=send_sems_l.at[slot], recv_sem=recv_sems_l.at[slot],
                device_id=(left,), device_id_type=MESH,
            )
            rdma_r.start()
            rdma_l.start()
            rdma_r.wait()
            rdma_l.wait()
            pl.semaphore_signal(credit_r, inc=1, device_id=(left,),
                                device_id_type=MESH)
            pl.semaphore_signal(credit_l, inc=1, device_id=(right,),
                                device_id_type=MESH)
            upd_r = (acc_ref[pl.ds(recv_r * CH, CH), pl.ds(0, HN)]
                     + rbuf_r[slot].astype(jnp.float32))
            upd_l = (acc_ref[pl.ds(recv_l * CH, CH), pl.ds(HN, HN)]
                     + rbuf_l[slot].astype(jnp.float32))
            if h < N_DEV - 2:
                sbuf_r[nslot] = upd_r.astype(jnp.bfloat16)
                sbuf_l[nslot] = upd_l.astype(jnp.bfloat16)
            acc_ref[pl.ds(recv_r * CH, CH), pl.ds(0, HN)] = upd_r
            acc_ref[pl.ds(recv_l * CH, CH), pl.ds(HN, HN)] = upd_l
        pl.semaphore_wait(credit_r, 2)
        pl.semaphore_wait(credit_l, 2)

        o_r = lax.rem(my + 1, N_DEV)
        o_l = lax.rem(my - 1 + N_DEV, N_DEV)
        z_r = acc_ref[pl.ds(o_r * CH, CH), pl.ds(0, HN)]
        z_l = acc_ref[pl.ds(o_l * CH, CH), pl.ds(HN, HN)]
        out_ref[pl.ds(o_r * CH, CH), pl.ds(0, HN)] = (
            z_r * (1.0 / (1.0 + jnp.exp(-z_r)))
        ).astype(out_ref.dtype)
        out_ref[pl.ds(o_l * CH, CH), pl.ds(HN, HN)] = (
            z_l * (1.0 / (1.0 + jnp.exp(-z_l)))
        ).astype(out_ref.dtype)

        for h in range(N_DEV - 1):
            slot = h % 2
            send_r = lax.rem(my + 1 - h + 2 * N_DEV, N_DEV)
            send_l = lax.rem(my - 1 + h + N_DEV, N_DEV)
            if h >= 2:
                pl.semaphore_wait(credit_r, 1)
                pl.semaphore_wait(credit_l, 1)
            rdma_r = pltpu.make_async_remote_copy(
                src_ref=out_ref.at[pl.ds(send_r * CH, CH), pl.ds(0, HN)],
                dst_ref=out_ref.at[pl.ds(send_r * CH, CH), pl.ds(0, HN)],
                send_sem=send_sems_r.at[slot], recv_sem=recv_sems_r.at[slot],
                device_id=(right,), device_id_type=MESH,
            )
            rdma_l = pltpu.make_async_remote_copy(
                src_ref=out_ref.at[pl.ds(send_l * CH, CH), pl.ds(HN, HN)],
                dst_ref=out_ref.at[pl.ds(send_l * CH, CH), pl.ds(HN, HN)],
                send_sem=send_sems_l.at[slot], recv_sem=recv_sems_l.at[slot],
                device_id=(left,), device_id_type=MESH,
            )
            rdma_r.start()
            rdma_l.start()
            rdma_r.wait()
            rdma_l.wait()
            pl.semaphore_signal(credit_r, inc=1, device_id=(left,),
                                device_id_type=MESH)
            pl.semaphore_signal(credit_l, inc=1, device_id=(right,),
                                device_id_type=MESH)
        pl.semaphore_wait(credit_r, 2)
        pl.semaphore_wait(credit_l, 2)

    return pl.pallas_call(
        body,
        out_shape=jax.ShapeDtypeStruct((M, N), jnp.bfloat16),
        in_specs=[
            pl.BlockSpec(memory_space=pltpu.VMEM),
            pl.BlockSpec(memory_space=pltpu.VMEM),
        ],
        out_specs=pl.BlockSpec(memory_space=pltpu.VMEM),
        scratch_shapes=[
            pltpu.VMEM((M, N), jnp.float32),
            pltpu.VMEM((2, CH, HN), jnp.bfloat16),
            pltpu.VMEM((2, CH, HN), jnp.bfloat16),
            pltpu.VMEM((2, CH, HN), jnp.bfloat16),
            pltpu.VMEM((2, CH, HN), jnp.bfloat16),
            pltpu.SemaphoreType.DMA((2,)),
            pltpu.SemaphoreType.DMA((2,)),
            pltpu.SemaphoreType.DMA((2,)),
            pltpu.SemaphoreType.DMA((2,)),
            pltpu.SemaphoreType.REGULAR,
            pltpu.SemaphoreType.REGULAR,
        ],
        compiler_params=pltpu.CompilerParams(collective_id=0),
    )(A, B)


# device time: 167412 ns/iter; 1.9949x vs baseline; 1.2456x over previous
import jax
import jax.numpy as jnp
from jax import lax
from jax.experimental import pallas as pl
from jax.experimental.pallas import tpu as pltpu

N_DEV = 16
M = 2048
N = 2048
CH = M // N_DEV
HN = N // 2

MESH = pl.DeviceIdType.MESH

RING = [0, 4, 8, 12, 13, 9, 5, 1, 2, 6, 10, 14, 15, 11, 7, 3]
INV = [0] * N_DEV
for _p, _l in enumerate(RING):
    INV[_l] = _p


def kernel(A, B):
    A = A.astype(jnp.bfloat16)
    B = B.astype(jnp.bfloat16)

    my = lax.axis_index("i")
    ring = jnp.array(RING, dtype=jnp.int32)
    p = jnp.array(INV, dtype=jnp.int32)[my]
    right = ring[lax.rem(p + 1, N_DEV)]
    left = ring[lax.rem(p - 1 + N_DEV, N_DEV)]
    scalars = jnp.stack([p, left, right]).astype(jnp.int32)

    def silu(z):
        return z * (1.0 / (1.0 + jnp.exp(-z)))

    def body(s_ref, a_ref, b_ref, out_ref,
             sbuf_r, rbuf_r, sbuf_l, rbuf_l,
             send_sems_r, recv_sems_r, send_sems_l, recv_sems_l,
             credit_r, credit_l):
        p = s_ref[0]
        left = s_ref[1]
        right = s_ref[2]

        barrier = pltpu.get_barrier_semaphore()
        for nbr in (left, right):
            pl.semaphore_signal(barrier, inc=1, device_id=(nbr,),
                                device_id_type=MESH)
        pl.semaphore_wait(barrier, 2)

        def part(idx, col0):
            return jnp.dot(
                a_ref[pl.ds(idx * CH, CH), :],
                b_ref[:, pl.ds(col0, HN)],
                preferred_element_type=jnp.float32,
            )

        sbuf_r[0] = part(p, 0).astype(jnp.bfloat16)
        sbuf_l[0] = part(p, HN).astype(jnp.bfloat16)
        for h in range(N_DEV - 1):
            slot = h % 2
            nslot = (h + 1) % 2
            recv_r = lax.rem(p - h - 1 + 2 * N_DEV, N_DEV)
            recv_l = lax.rem(p + h + 1, N_DEV)
            if h >= 2:
                pl.semaphore_wait(credit_r, 1)
                pl.semaphore_wait(credit_l, 1)
            rdma_r = pltpu.make_async_remote_copy(
                src_ref=sbuf_r.at[slot], dst_ref=rbuf_r.at[slot],
                send_sem=send_sems_r.at[slot], recv_sem=recv_sems_r.at[slot],
                device_id=(right,), device_id_type=MESH,
            )
            rdma_l = pltpu.make_async_remote_copy(
                src_ref=sbuf_l.at[slot], dst_ref=rbuf_l.at[slot],
                send_sem=send_sems_l.at[slot], recv_sem=recv_sems_l.at[slot],
                device_id=(left,), device_id_type=MESH,
            )
            rdma_r.start()
            rdma_l.start()
            part_r = part(recv_r, 0)
            part_l = part(recv_l, HN)
            rdma_r.wait()
            rdma_l.wait()
            pl.semaphore_signal(credit_r, inc=1, device_id=(left,),
                                device_id_type=MESH)
            pl.semaphore_signal(credit_l, inc=1, device_id=(right,),
                                device_id_type=MESH)
            upd_r = part_r + rbuf_r[slot].astype(jnp.float32)
            upd_l = part_l + rbuf_l[slot].astype(jnp.float32)
            if h < N_DEV - 2:
                sbuf_r[nslot] = upd_r.astype(jnp.bfloat16)
                sbuf_l[nslot] = upd_l.astype(jnp.bfloat16)
            else:
                out_ref[pl.ds(recv_r * CH, CH), pl.ds(0, HN)] = (
                    silu(upd_r).astype(out_ref.dtype)
                )
                out_ref[pl.ds(recv_l * CH, CH), pl.ds(HN, HN)] = (
                    silu(upd_l).astype(out_ref.dtype)
                )
        pl.semaphore_wait(credit_r, 2)
        pl.semaphore_wait(credit_l, 2)

        for h in range(N_DEV - 1):
            slot = h % 2
            send_r = lax.rem(p + 1 - h + 2 * N_DEV, N_DEV)
            send_l = lax.rem(p - 1 + h + N_DEV, N_DEV)
            if h >= 2:
                pl.semaphore_wait(credit_r, 1)
                pl.semaphore_wait(credit_l, 1)
            rdma_r = pltpu.make_async_remote_copy(
                src_ref=out_ref.at[pl.ds(send_r * CH, CH), pl.ds(0, HN)],
                dst_ref=out_ref.at[pl.ds(send_r * CH, CH), pl.ds(0, HN)],
                send_sem=send_sems_r.at[slot], recv_sem=recv_sems_r.at[slot],
                device_id=(right,), device_id_type=MESH,
            )
            rdma_l = pltpu.make_async_remote_copy(
                src_ref=out_ref.at[pl.ds(send_l * CH, CH), pl.ds(HN, HN)],
                dst_ref=out_ref.at[pl.ds(send_l * CH, CH), pl.ds(HN, HN)],
                send_sem=send_sems_l.at[slot], recv_sem=recv_sems_l.at[slot],
                device_id=(left,), device_id_type=MESH,
            )
            rdma_r.start()
            rdma_l.start()
            rdma_r.wait()
            rdma_l.wait()
            pl.semaphore_signal(credit_r, inc=1, device_id=(left,),
                                device_id_type=MESH)
            pl.semaphore_signal(credit_l, inc=1, device_id=(right,),
                                device_id_type=MESH)
        pl.semaphore_wait(credit_r, 2)
        pl.semaphore_wait(credit_l, 2)

    return pl.pallas_call(
        body,
        out_shape=jax.ShapeDtypeStruct((M, N), jnp.bfloat16),
        in_specs=[
            pl.BlockSpec(memory_space=pltpu.SMEM),
            pl.BlockSpec(memory_space=pltpu.VMEM),
            pl.BlockSpec(memory_space=pltpu.VMEM),
        ],
        out_specs=pl.BlockSpec(memory_space=pltpu.VMEM),
        scratch_shapes=[
            pltpu.VMEM((2, CH, HN), jnp.bfloat16),
            pltpu.VMEM((2, CH, HN), jnp.bfloat16),
            pltpu.VMEM((2, CH, HN), jnp.bfloat16),
            pltpu.VMEM((2, CH, HN), jnp.bfloat16),
            pltpu.SemaphoreType.DMA((2,)),
            pltpu.SemaphoreType.DMA((2,)),
            pltpu.SemaphoreType.DMA((2,)),
            pltpu.SemaphoreType.DMA((2,)),
            pltpu.SemaphoreType.REGULAR,
            pltpu.SemaphoreType.REGULAR,
        ],
        compiler_params=pltpu.CompilerParams(collective_id=0),
    )(scalars, A, B)


# device time: 121096 ns/iter; 2.7578x vs baseline; 1.3825x over previous
import jax
import jax.numpy as jnp
from jax import lax
from jax.experimental import pallas as pl
from jax.experimental.pallas import tpu as pltpu

N_DEV = 16
M = 2048
N = 2048
CH = M // N_DEV
SB = CH // 2
HN = N // 2

MESH = pl.DeviceIdType.MESH

RING = [0, 4, 8, 12, 13, 9, 5, 1, 2, 6, 10, 14, 15, 11, 7, 3]
INV = [0] * N_DEV
for _p, _l in enumerate(RING):
    INV[_l] = _p


def kernel(A, B):
    A = A.astype(jnp.bfloat16)
    B = B.astype(jnp.bfloat16)

    my = lax.axis_index("i")
    ring = jnp.array(RING, dtype=jnp.int32)
    p = jnp.array(INV, dtype=jnp.int32)[my]
    right = ring[lax.rem(p + 1, N_DEV)]
    left = ring[lax.rem(p - 1 + N_DEV, N_DEV)]
    scalars = jnp.stack([p, left, right]).astype(jnp.int32)

    def silu(z):
        return z * (1.0 / (1.0 + jnp.exp(-z)))

    def body(s_ref, a_ref, b_ref, out_ref,
             sbufs, rbufs, send_sems, recv_sems, credits):
        p = s_ref[0]
        left = s_ref[1]
        right = s_ref[2]

        barrier = pltpu.get_barrier_semaphore()
        for nbr in (left, right):
            pl.semaphore_signal(barrier, inc=1, device_id=(nbr,),
                                device_id_type=MESH)
        pl.semaphore_wait(barrier, 2)

        def part(idx, col0):
            return jnp.dot(
                a_ref[pl.ds(idx * CH, CH), :],
                b_ref[:, pl.ds(col0, HN)],
                preferred_element_type=jnp.float32,
            )

        def stream_dir(s):
            return (right, left) if s < 2 else (left, right)

        def rs_rdma(s, slot):
            dst, _ = stream_dir(s)
            return pltpu.make_async_remote_copy(
                src_ref=sbufs.at[s, slot], dst_ref=rbufs.at[s, slot],
                send_sem=send_sems.at[s, slot], recv_sem=recv_sems.at[s, slot],
                device_id=(dst,), device_id_type=MESH,
            )

        own_r = part(p, 0)
        own_l = part(p, HN)
        sbufs[0, 0] = own_r[:SB].astype(jnp.bfloat16)
        sbufs[1, 0] = own_r[SB:].astype(jnp.bfloat16)
        sbufs[2, 0] = own_l[:SB].astype(jnp.bfloat16)
        sbufs[3, 0] = own_l[SB:].astype(jnp.bfloat16)
        for s in range(4):
            rs_rdma(s, 0).start()
        part_r = part(lax.rem(p - 1 + N_DEV, N_DEV), 0)
        part_l = part(lax.rem(p + 1, N_DEV), HN)

        for h in range(N_DEV - 1):
            slot = h % 2
            nslot = (h + 1) % 2
            recv_r = lax.rem(p - h - 1 + 2 * N_DEV, N_DEV)
            recv_l = lax.rem(p + h + 1, N_DEV)
            last = h == N_DEV - 2
            for s in range(4):
                _, up = stream_dir(s)
                prt = part_r if s < 2 else part_l
                ridx = recv_r if s < 2 else recv_l
                col0 = 0 if s < 2 else HN
                rows = slice(0, SB) if s % 2 == 0 else slice(SB, CH)
                rs_rdma(s, slot).wait()
                upd = prt[rows] + rbufs[s, slot].astype(jnp.float32)
                if not last:
                    sbufs[s, nslot] = upd.astype(jnp.bfloat16)
                    pl.semaphore_signal(credits.at[s], inc=1, device_id=(up,),
                                        device_id_type=MESH)
                    if h >= 1:
                        pl.semaphore_wait(credits.at[s], 1)
                    rs_rdma(s, nslot).start()
                else:
                    r0 = ridx * CH + (0 if s % 2 == 0 else SB)
                    out_ref[pl.ds(r0, SB), pl.ds(col0, HN)] = (
                        silu(upd).astype(out_ref.dtype)
                    )
                    pl.semaphore_signal(credits.at[s], inc=1, device_id=(up,),
                                        device_id_type=MESH)
            if not last:
                part_r = part(lax.rem(p - h - 2 + 2 * N_DEV, N_DEV), 0)
                part_l = part(lax.rem(p + h + 2, N_DEV), HN)
        for s in range(4):
            pl.semaphore_wait(credits.at[s], 2)

        def ag_rdma(s, slot, cidx):
            dst, _ = stream_dir(s)
            col0 = 0 if s < 2 else HN
            r0 = cidx * CH + (0 if s % 2 == 0 else SB)
            return pltpu.make_async_remote_copy(
                src_ref=out_ref.at[pl.ds(r0, SB), pl.ds(col0, HN)],
                dst_ref=out_ref.at[pl.ds(r0, SB), pl.ds(col0, HN)],
                send_sem=send_sems.at[s, slot], recv_sem=recv_sems.at[s, slot],
                device_id=(dst,), device_id_type=MESH,
            )

        o_r = lax.rem(p + 1, N_DEV)
        o_l = lax.rem(p - 1 + N_DEV, N_DEV)
        for s in range(4):
            ag_rdma(s, 0, o_r if s < 2 else o_l).start()
        for h in range(N_DEV - 1):
            slot = h % 2
            nslot = (h + 1) % 2
            send_r = lax.rem(p - h + 2 * N_DEV, N_DEV)
            send_l = lax.rem(p + h, N_DEV)
            last = h == N_DEV - 2
            for s in range(4):
                _, up = stream_dir(s)
                cidx = send_r if s < 2 else send_l
                ag_rdma(s, slot, cidx).wait()
                pl.semaphore_signal(credits.at[s], inc=1, device_id=(up,),
                                    device_id_type=MESH)
                if not last:
                    if h >= 1:
                        pl.semaphore_wait(credits.at[s], 1)
                    ag_rdma(s, nslot, cidx).start()
        for s in range(4):
            pl.semaphore_wait(credits.at[s], 2)

    return pl.pallas_call(
        body,
        out_shape=jax.ShapeDtypeStruct((M, N), jnp.bfloat16),
        in_specs=[
            pl.BlockSpec(memory_space=pltpu.SMEM),
            pl.BlockSpec(memory_space=pltpu.VMEM),
            pl.BlockSpec(memory_space=pltpu.VMEM),
        ],
        out_specs=pl.BlockSpec(memory_space=pltpu.VMEM),
        scratch_shapes=[
            pltpu.VMEM((4, 2, SB, HN), jnp.bfloat16),
            pltpu.VMEM((4, 2, SB, HN), jnp.bfloat16),
            pltpu.SemaphoreType.DMA((4, 2)),
            pltpu.SemaphoreType.DMA((4, 2)),
            pltpu.SemaphoreType.REGULAR((4,)),
        ],
        compiler_params=pltpu.CompilerParams(collective_id=0),
    )(scalars, A, B)


# device time: 108660 ns/iter; 3.0735x vs baseline; 1.1144x over previous
import jax
import jax.numpy as jnp
from jax import lax
from jax.experimental import pallas as pl
from jax.experimental.pallas import tpu as pltpu

N_DEV = 16
M = 2048
N = 2048
CH = M // N_DEV
NSUB = 4
NS = 2 * NSUB
SB = CH // NSUB
HN = N // 2

MESH = pl.DeviceIdType.MESH

RING = [0, 4, 8, 12, 13, 9, 5, 1, 2, 6, 10, 14, 15, 11, 7, 3]
INV = [0] * N_DEV
for _p, _l in enumerate(RING):
    INV[_l] = _p


def kernel(A, B):
    A = A.astype(jnp.bfloat16)
    B = B.astype(jnp.bfloat16)

    my = lax.axis_index("i")
    ring = jnp.array(RING, dtype=jnp.int32)
    p = jnp.array(INV, dtype=jnp.int32)[my]
    right = ring[lax.rem(p + 1, N_DEV)]
    left = ring[lax.rem(p - 1 + N_DEV, N_DEV)]
    scalars = jnp.stack([p, left, right]).astype(jnp.int32)

    def silu(z):
        return z * (1.0 / (1.0 + jnp.exp(-z)))

    def body(s_ref, a_ref, b_ref, out_ref,
             sbufs, rbufs, send_sems, recv_sems, credits):
        p = s_ref[0]
        left = s_ref[1]
        right = s_ref[2]

        barrier = pltpu.get_barrier_semaphore()
        for nbr in (left, right):
            pl.semaphore_signal(barrier, inc=1, device_id=(nbr,),
                                device_id_type=MESH)
        pl.semaphore_wait(barrier, 2)

        def part(idx, col0):
            return jnp.dot(
                a_ref[pl.ds(idx * CH, CH), :],
                b_ref[:, pl.ds(col0, HN)],
                preferred_element_type=jnp.float32,
            )

        def stream_dir(s):
            return (right, left) if s % 2 == 0 else (left, right)

        def rows_of(s):
            sub = s // 2
            return slice(sub * SB, (sub + 1) * SB)

        def rs_rdma(s, slot):
            dst, _ = stream_dir(s)
            return pltpu.make_async_remote_copy(
                src_ref=sbufs.at[s, slot], dst_ref=rbufs.at[s, slot],
                send_sem=send_sems.at[s, slot], recv_sem=recv_sems.at[s, slot],
                device_id=(dst,), device_id_type=MESH,
            )

        own_r = part(p, 0)
        own_l = part(p, HN)
        for s in range(NS):
            own = own_r if s % 2 == 0 else own_l
            sbufs[s, 0] = own[rows_of(s)].astype(jnp.bfloat16)
        for s in range(NS):
            rs_rdma(s, 0).start()
        part_r = part(lax.rem(p - 1 + N_DEV, N_DEV), 0)
        part_l = part(lax.rem(p + 1, N_DEV), HN)

        for h in range(N_DEV - 1):
            slot = h % 2
            nslot = (h + 1) % 2
            recv_r = lax.rem(p - h - 1 + 2 * N_DEV, N_DEV)
            recv_l = lax.rem(p + h + 1, N_DEV)
            last = h == N_DEV - 2
            for s in range(NS):
                _, up = stream_dir(s)
                cw = s % 2 == 0
                prt = part_r if cw else part_l
                ridx = recv_r if cw else recv_l
                col0 = 0 if cw else HN
                rows = rows_of(s)
                rs_rdma(s, slot).wait()
                upd = prt[rows] + rbufs[s, slot].astype(jnp.float32)
                if not last:
                    sbufs[s, nslot] = upd.astype(jnp.bfloat16)
                    pl.semaphore_signal(credits.at[s], inc=1, device_id=(up,),
                                        device_id_type=MESH)
                    if h >= 1:
                        pl.semaphore_wait(credits.at[s], 1)
                    rs_rdma(s, nslot).start()
                else:
                    r0 = ridx * CH + (s // 2) * SB
                    out_ref[pl.ds(r0, SB), pl.ds(col0, HN)] = (
                        silu(upd).astype(out_ref.dtype)
                    )
                    pl.semaphore_signal(credits.at[s], inc=1, device_id=(up,),
                                        device_id_type=MESH)
            if not last:
                part_r = part(lax.rem(p - h - 2 + 2 * N_DEV, N_DEV), 0)
                part_l = part(lax.rem(p + h + 2, N_DEV), HN)
        for s in range(NS):
            pl.semaphore_wait(credits.at[s], 2)

        def ag_rdma(s, slot, cidx):
            dst, _ = stream_dir(s)
            col0 = 0 if s % 2 == 0 else HN
            r0 = cidx * CH + (s // 2) * SB
            return pltpu.make_async_remote_copy(
                src_ref=out_ref.at[pl.ds(r0, SB), pl.ds(col0, HN)],
                dst_ref=out_ref.at[pl.ds(r0, SB), pl.ds(col0, HN)],
                send_sem=send_sems.at[s, slot], recv_sem=recv_sems.at[s, slot],
                device_id=(dst,), device_id_type=MESH,
            )

        o_r = lax.rem(p + 1, N_DEV)
        o_l = lax.rem(p - 1 + N_DEV, N_DEV)
        for s in range(NS):
            ag_rdma(s, 0, o_r if s % 2 == 0 else o_l).start()
        for h in range(N_DEV - 1):
            slot = h % 2
            nslot = (h + 1) % 2
            send_r = lax.rem(p - h + 2 * N_DEV, N_DEV)
            send_l = lax.rem(p + h, N_DEV)
            last = h == N_DEV - 2
            for s in range(NS):
                _, up = stream_dir(s)
                cidx = send_r if s % 2 == 0 else send_l
                ag_rdma(s, slot, cidx).wait()
                pl.semaphore_signal(credits.at[s], inc=1, device_id=(up,),
                                    device_id_type=MESH)
                if not last:
                    if h >= 1:
                        pl.semaphore_wait(credits.at[s], 1)
                    ag_rdma(s, nslot, cidx).start()
        for s in range(NS):
            pl.semaphore_wait(credits.at[s], 2)

    return pl.pallas_call(
        body,
        out_shape=jax.ShapeDtypeStruct((M, N), jnp.bfloat16),
        in_specs=[
            pl.BlockSpec(memory_space=pltpu.SMEM),
            pl.BlockSpec(memory_space=pltpu.VMEM),
            pl.BlockSpec(memory_space=pltpu.VMEM),
        ],
        out_specs=pl.BlockSpec(memory_space=pltpu.VMEM),
        scratch_shapes=[
            pltpu.VMEM((NS, 2, SB, HN), jnp.bfloat16),
            pltpu.VMEM((NS, 2, SB, HN), jnp.bfloat16),
            pltpu.SemaphoreType.DMA((NS, 2)),
            pltpu.SemaphoreType.DMA((NS, 2)),
            pltpu.SemaphoreType.REGULAR((NS,)),
        ],
        compiler_params=pltpu.CompilerParams(collective_id=0),
    )(scalars, A, B)
